# Initial kernel scaffold; baseline (speedup 1.0000x reference)
import jax
import jax.numpy as jnp
from jax.experimental import pallas as pl


def kernel(seg_confidences, spatial_features, rule_weights):
    raise NotImplementedError("write your pallas kernel here")


# speedup vs baseline: 525.9214x; 525.9214x over previous
import functools

import jax
import jax.numpy as jnp
import numpy as np
from jax import lax
from jax.experimental import pallas as pl
from jax.experimental.pallas import tpu as pltpu
from jax.experimental.pallas import tpu_sc as plsc

_N_SEG = 100
_N_SPAT = 50
_N_OUT = 100
_B = 128
_NB = 112
_SEG_W = 112
_SPAT_W = 64
_N_GROUPS = 7
_ROWS_PER_TILE = 4


def _build_tables():
    members = [[] for _ in range(_NB)]
    for a in range(_N_SEG):
        for b in range(_N_SPAT):
            s = (a * b) // 10
            if s < _N_OUT:
                members[s].append((a, b))
    counts = [len(m) for m in members]
    slots = tuple(int(max(counts[g * 16:(g + 1) * 16])) for g in range(_N_GROUPS))
    s1_rows, s2_rows = [], []
    for g in range(_N_GROUPS):
        for m in range(slots[g]):
            r1, r2 = [], []
            for lane in range(16):
                bkt = g * 16 + lane
                if m < counts[bkt]:
                    a, b = members[bkt][m]
                else:
                    a, b = _N_SEG, _N_SPAT
                r1.append(a)
                r2.append(b)
            s1_rows.append(r1)
            s2_rows.append(r2)
    return slots, np.asarray(s1_rows, np.int32), np.asarray(s2_rows, np.int32)


_SLOTS, _S1_TAB, _S2_TAB = _build_tables()
_N_SLOTS = int(sum(_SLOTS))


def _sc_body(seg_hbm, spat_hbm, s1_hbm, s2_hbm, out_hbm,
             seg_v, spat_v, s1_v, s2_v, out_v):
    wid = lax.axis_index("s") * 2 + lax.axis_index("c")
    base = wid * _ROWS_PER_TILE
    pltpu.sync_copy(seg_hbm.at[pl.ds(base * _SEG_W, _ROWS_PER_TILE * _SEG_W)], seg_v)
    pltpu.sync_copy(spat_hbm.at[pl.ds(base * _SPAT_W, _ROWS_PER_TILE * _SPAT_W)], spat_v)
    pltpu.sync_copy(s1_hbm, s1_v)
    pltpu.sync_copy(s2_hbm, s2_v)

    one = jnp.float32(1.0)

    def row_body(r, carry):
        seg_off = r * _SEG_W
        spat_off = r * _SPAT_W
        slot = 0
        for g in range(_N_GROUPS):
            t0 = jnp.zeros((16,), jnp.float32)
            t1 = jnp.zeros((16,), jnp.float32)
            t2 = jnp.zeros((16,), jnp.float32)
            for _ in range(_SLOTS[g]):
                c1 = s1_v[pl.ds(slot * 16, 16)] + seg_off
                c2 = s2_v[pl.ds(slot * 16, 16)] + spat_off
                gs = plsc.load_gather(seg_v, [c1])
                gp = plsc.load_gather(spat_v, [c2])
                v = gs * gp
                a0 = jnp.maximum(t0, v)
                b0 = jnp.minimum(t0, v)
                a1 = jnp.maximum(t1, b0)
                b1 = jnp.minimum(t1, b0)
                t2 = jnp.maximum(t2, b1)
                t1 = a1
                t0 = a0
                slot += 1
            res = one - (one - t0) * (one - t1) * (one - t2)
            out_v[pl.ds(r * _NB + g * 16, 16)] = res
        return carry

    lax.fori_loop(0, _ROWS_PER_TILE, row_body, 0)
    pltpu.sync_copy(out_v, out_hbm.at[pl.ds(base * _NB, _ROWS_PER_TILE * _NB)])


@jax.jit
def _run(seg_flat, spat_flat, s1_flat, s2_flat):
    mesh = plsc.VectorSubcoreMesh(core_axis_name="c", subcore_axis_name="s")
    fn = functools.partial(
        pl.kernel,
        mesh=mesh,
        out_type=jax.ShapeDtypeStruct((_B * _NB,), jnp.float32),
        compiler_params=pltpu.CompilerParams(needs_layout_passes=False),
        scratch_types=[
            pltpu.VMEM((_ROWS_PER_TILE * _SEG_W,), jnp.float32),
            pltpu.VMEM((_ROWS_PER_TILE * _SPAT_W,), jnp.float32),
            pltpu.VMEM((_N_SLOTS * 16,), jnp.int32),
            pltpu.VMEM((_N_SLOTS * 16,), jnp.int32),
            pltpu.VMEM((_ROWS_PER_TILE * _NB,), jnp.float32),
        ],
    )(_sc_body)
    return fn(seg_flat, spat_flat, s1_flat, s2_flat)


def kernel(seg_confidences, spatial_features, rule_weights):
    del rule_weights
    seg_pad = jnp.concatenate(
        [seg_confidences,
         jnp.zeros((_B, _SEG_W - _N_SEG), jnp.float32)], axis=1)
    spat_pad = jnp.concatenate(
        [spatial_features,
         jnp.zeros((_B, _SPAT_W - _N_SPAT), jnp.float32)], axis=1)
    out = _run(seg_pad.reshape(-1), spat_pad.reshape(-1),
               jnp.asarray(_S1_TAB).reshape(-1), jnp.asarray(_S2_TAB).reshape(-1))
    return out.reshape(_B, _NB)[:, :_N_OUT]


# speedup vs baseline: 640.1251x; 1.2171x over previous
import functools

import jax
import jax.numpy as jnp
import numpy as np
from jax import lax
from jax.experimental import pallas as pl
from jax.experimental.pallas import tpu as pltpu
from jax.experimental.pallas import tpu_sc as plsc

_N_SEG = 100
_N_SPAT = 50
_N_OUT = 100
_B = 128
_NB = 112
_SEG_W = 112
_SPAT_W = 64
_N_GROUPS = 7
_ROWS_PER_TILE = 4


def _build_tables():
    members = [[] for _ in range(_N_OUT)]
    for a in range(_N_SEG):
        for b in range(_N_SPAT):
            s = (a * b) // 10
            if s < _N_OUT:
                members[s].append((a, b))
    counts = [len(m) for m in members]

    b0 = members[0]
    b0_slots = (len(b0) + 15) // 16
    b0_pad = b0 + [(_N_SEG, _N_SPAT)] * (b0_slots * 16 - len(b0))

    rem = sorted(range(1, _N_OUT), key=lambda s: -counts[s])
    groups, perms = [], []
    res0_lane = None
    trash = _N_OUT
    for g in range(_N_GROUPS):
        lanes = list(rem[g * 16:(g + 1) * 16])
        perm = list(lanes)
        while len(perm) < 16:
            if g == _N_GROUPS - 1 and res0_lane is None:
                res0_lane = len(perm)
                perm.append(0)
            else:
                perm.append(trash)
                trash += 1
            lanes.append(None)
        groups.append(lanes)
        perms.append(perm)
    gslots = [max(counts[s] for s in g if s is not None) for g in groups]

    chains = [[[b0_pad[m * 16 + l] for l in range(16)]
               for m in range(b0_slots)]]
    for g in range(_N_GROUPS):
        rows = []
        for m in range(gslots[g]):
            row = []
            for l in range(16):
                s = groups[g][l]
                if s is not None and m < counts[s]:
                    row.append(members[s][m])
                else:
                    row.append((_N_SEG, _N_SPAT))
            rows.append(row)
        chains.append(rows)
    chain_lens = tuple(len(c) for c in chains)
    s1 = np.array([[p[0] for p in row] for c in chains for row in c], np.int32)
    s2 = np.array([[p[1] for p in row] for c in chains for row in c], np.int32)
    return chain_lens, res0_lane, s1, s2, np.array(perms, np.int32)


_CHAIN_LENS, _RES0_LANE, _S1_TAB, _S2_TAB, _PERM_TAB = _build_tables()
_CHAIN_OFF = tuple(int(x) for x in np.cumsum((0,) + _CHAIN_LENS))
_N_SLOTS = int(sum(_CHAIN_LENS))
_LANE_IOTA = np.arange(16, dtype=np.int32)


def _sc_body(seg_hbm, spat_hbm, s1_hbm, s2_hbm, perm_hbm, out_hbm,
             seg_v, spat_v, s1_v, s2_v, perm_v, out_v):
    wid = lax.axis_index("s") * 2 + lax.axis_index("c")
    base = wid * _ROWS_PER_TILE
    pltpu.sync_copy(seg_hbm.at[pl.ds(base * _SEG_W, _ROWS_PER_TILE * _SEG_W)], seg_v)
    pltpu.sync_copy(spat_hbm.at[pl.ds(base * _SPAT_W, _ROWS_PER_TILE * _SPAT_W)], spat_v)
    pltpu.sync_copy(s1_hbm, s1_v)
    pltpu.sync_copy(s2_hbm, s2_v)
    pltpu.sync_copy(perm_hbm, perm_v)

    one = jnp.float32(1.0)
    zero16 = jnp.zeros((16,), jnp.float32)
    iota = lax.iota(jnp.int32, 16)
    n_chains = len(_CHAIN_LENS)

    def row_body(r, carry):
        seg_off = r * _SEG_W
        spat_off = r * _SPAT_W
        ts = [[zero16, zero16, zero16] for _ in range(n_chains)]
        for m in range(max(_CHAIN_LENS)):
            for c in range(n_chains):
                if m >= _CHAIN_LENS[c]:
                    continue
                slot = _CHAIN_OFF[c] + m
                c1 = s1_v[pl.ds(slot * 16, 16)] + seg_off
                c2 = s2_v[pl.ds(slot * 16, 16)] + spat_off
                v = plsc.load_gather(seg_v, [c1]) * plsc.load_gather(spat_v, [c2])
                t0, t1, t2 = ts[c]
                a0 = jnp.maximum(t0, v)
                b0 = jnp.minimum(t0, v)
                a1 = jnp.maximum(t1, b0)
                b1 = jnp.minimum(t1, b0)
                ts[c] = [a0, a1, jnp.maximum(t2, b1)]

        u0, u1, u2 = ts[0]
        acc = one
        for _ in range(3):
            mx = jnp.max(u0)
            acc = acc * (one - mx)
            msk = iota == plsc.all_reduce_ffs(u0 == mx)
            u0 = jnp.where(msk, u1, u0)
            u1 = jnp.where(msk, u2, u1)
            u2 = jnp.where(msk, zero16, u2)
        res0 = one - acc

        for g in range(_N_GROUPS):
            t0, t1, t2 = ts[1 + g]
            res = one - (one - t0) * (one - t1) * (one - t2)
            if g == _N_GROUPS - 1:
                res = jnp.where(iota == _RES0_LANE, res0, res)
            pos = perm_v[pl.ds(g * 16, 16)] + r * _NB
            plsc.store_scatter(out_v, [pos], res)
        return carry

    lax.fori_loop(0, _ROWS_PER_TILE, row_body, 0)
    pltpu.sync_copy(out_v, out_hbm.at[pl.ds(base * _NB, _ROWS_PER_TILE * _NB)])


@jax.jit
def _run(seg_flat, spat_flat, s1_flat, s2_flat, perm_flat):
    mesh = plsc.VectorSubcoreMesh(core_axis_name="c", subcore_axis_name="s")
    fn = functools.partial(
        pl.kernel,
        mesh=mesh,
        out_type=jax.ShapeDtypeStruct((_B * _NB,), jnp.float32),
        compiler_params=pltpu.CompilerParams(needs_layout_passes=False),
        scratch_types=[
            pltpu.VMEM((_ROWS_PER_TILE * _SEG_W,), jnp.float32),
            pltpu.VMEM((_ROWS_PER_TILE * _SPAT_W,), jnp.float32),
            pltpu.VMEM((_N_SLOTS * 16,), jnp.int32),
            pltpu.VMEM((_N_SLOTS * 16,), jnp.int32),
            pltpu.VMEM((_N_GROUPS * 16,), jnp.int32),
            pltpu.VMEM((_ROWS_PER_TILE * _NB,), jnp.float32),
        ],
    )(_sc_body)
    return fn(seg_flat, spat_flat, s1_flat, s2_flat, perm_flat)


def kernel(seg_confidences, spatial_features, rule_weights):
    del rule_weights
    seg_pad = jnp.concatenate(
        [seg_confidences,
         jnp.zeros((_B, _SEG_W - _N_SEG), jnp.float32)], axis=1)
    spat_pad = jnp.concatenate(
        [spatial_features,
         jnp.zeros((_B, _SPAT_W - _N_SPAT), jnp.float32)], axis=1)
    out = _run(seg_pad.reshape(-1), spat_pad.reshape(-1),
               jnp.asarray(_S1_TAB).reshape(-1), jnp.asarray(_S2_TAB).reshape(-1),
               jnp.asarray(_PERM_TAB).reshape(-1))
    return out.reshape(_B, _NB)[:, :_N_OUT]


# speedup vs baseline: 699.7444x; 1.0931x over previous
import functools

import jax
import jax.numpy as jnp
import numpy as np
from jax import lax
from jax.experimental import pallas as pl
from jax.experimental.pallas import tpu as pltpu
from jax.experimental.pallas import tpu_sc as plsc

_N_SEG = 100
_N_SPAT = 50
_N_OUT = 100
_B = 128
_SEG_W = 112
_SPAT_W = 64
_N_GROUPS = 7
_ROWS = 4
_U = 4


def _build_tables():
    members = [[] for _ in range(_N_OUT)]
    for a in range(_N_SEG):
        for b in range(_N_SPAT):
            s = (a * b) // 10
            if s < _N_OUT:
                members[s].append((a, b))
    counts = [len(m) for m in members]

    b0 = members[0]
    b0_slots = (len(b0) + 15) // 16
    b0_pad = b0 + [(_N_SEG, _N_SPAT)] * (b0_slots * 16 - len(b0))

    rem = sorted(range(1, _N_OUT), key=lambda s: -counts[s])
    group_buckets = [rem[g * 14:(g + 1) * 14] for g in range(6)] + [rem[84:99]]
    later_pool = list(rem[84:99])
    perms, groups = [], []
    res0_lane = None
    pool_i = 0
    for g in range(_N_GROUPS):
        lanes = list(group_buckets[g])
        perm = list(lanes)
        while len(perm) < 16:
            if g == _N_GROUPS - 1 and res0_lane is None:
                res0_lane = len(perm)
                perm.append(0)
            else:
                perm.append(later_pool[pool_i % len(later_pool)])
                pool_i += 1
            lanes.append(None)
        groups.append(lanes)
        perms.append(perm)
    gslots = [max(counts[s] for s in gg if s is not None) for gg in groups]

    def pad_u(n):
        return ((n + _U - 1) // _U) * _U

    chain_lens = tuple([pad_u(b0_slots)] + [pad_u(x) for x in gslots])

    chains = [[[b0_pad[m * 16 + l] if m < b0_slots else (_N_SEG, _N_SPAT)
                for l in range(16)] for m in range(chain_lens[0])]]
    for g in range(_N_GROUPS):
        rows = []
        for m in range(chain_lens[1 + g]):
            row = []
            for l in range(16):
                s = groups[g][l]
                if s is not None and m < counts[s]:
                    row.append(members[s][m])
                else:
                    row.append((_N_SEG, _N_SPAT))
            rows.append(row)
        chains.append(rows)
    pack = np.array([[p[0] + (p[1] << 7) for p in row]
                     for c in chains for row in c], np.int32)
    return chain_lens, res0_lane, pack, np.array(perms, np.int32)


_CHAIN_LENS, _RES0_LANE, _PACK_TAB, _PERM_TAB = _build_tables()
_CHAIN_OFF = tuple(int(x) for x in np.cumsum((0,) + _CHAIN_LENS))
_N_SLOTS = int(sum(_CHAIN_LENS))


def _sc_body(seg_hbm, spat_hbm, pack_hbm, perm_hbm, out_hbm,
             seg_raw, spat_raw, seg_v, spat_v, pack_v, perm_v, out_v):
    wid = lax.axis_index("s") * 2 + lax.axis_index("c")
    base = wid * _ROWS
    pltpu.sync_copy(seg_hbm.at[pl.ds(base, _ROWS)], seg_raw)
    pltpu.sync_copy(spat_hbm.at[pl.ds(base, _ROWS)], spat_raw)
    pltpu.sync_copy(pack_hbm, pack_v)
    pltpu.sync_copy(perm_hbm, perm_v)

    one = jnp.float32(1.0)
    zero16 = jnp.zeros((16,), jnp.float32)
    iota = lax.iota(jnp.int32, 16)

    for r in range(_ROWS):
        row_splat = jnp.full((16,), r, jnp.int32)
        for k in range(_SEG_W // 16):
            src = iota + (16 * k)
            valid = src < _N_SEG
            g = plsc.load_gather(seg_raw, [row_splat, jnp.minimum(src, _N_SEG - 1)])
            seg_v[pl.ds(r * _SEG_W + 16 * k, 16)] = jnp.where(valid, g, zero16)
        for k in range(_SPAT_W // 16):
            src = iota + (16 * k)
            valid = src < _N_SPAT
            g = plsc.load_gather(spat_raw, [row_splat, jnp.minimum(src, _N_SPAT - 1)])
            spat_v[pl.ds(r * _SPAT_W + 16 * k, 16)] = jnp.where(valid, g, zero16)

    def row_body(r, carry):
        seg_off = r * _SEG_W
        spat_off = r * _SPAT_W

        def make_chain(c):
            cbase = _CHAIN_OFF[c] * 16

            def mbody(m, t):
                t0, t1, t2 = t
                moff = m * (_U * 16)
                for u in range(_U):
                    pk = pack_v[pl.ds(moff + (cbase + u * 16), 16)]
                    c1 = (pk & 127) + seg_off
                    c2 = lax.shift_right_logical(pk, 7) + spat_off
                    v = (plsc.load_gather(seg_v, [c1])
                         * plsc.load_gather(spat_v, [c2]))
                    a0 = jnp.maximum(t0, v)
                    b0 = jnp.minimum(t0, v)
                    a1 = jnp.maximum(t1, b0)
                    b1 = jnp.minimum(t1, b0)
                    t0, t1, t2 = a0, a1, jnp.maximum(t2, b1)
                return (t0, t1, t2)

            return lax.fori_loop(0, _CHAIN_LENS[c] // _U, mbody,
                                 (zero16, zero16, zero16))

        ts = [make_chain(c) for c in range(1 + _N_GROUPS)]

        u0, u1, u2 = ts[0]
        acc = one
        for _ in range(3):
            mx = jnp.max(u0)
            acc = acc * (one - mx)
            msk = iota == plsc.all_reduce_ffs(u0 == mx)
            u0 = jnp.where(msk, u1, u0)
            u1 = jnp.where(msk, u2, u1)
            u2 = jnp.where(msk, zero16, u2)
        res0 = one - acc

        row_splat = jnp.full((16,), 0, jnp.int32) + r
        for g in range(_N_GROUPS):
            t0, t1, t2 = ts[1 + g]
            res = one - (one - t0) * (one - t1) * (one - t2)
            if g == _N_GROUPS - 1:
                res = jnp.where(iota == _RES0_LANE, res0, res)
            pos = perm_v[pl.ds(g * 16, 16)]
            plsc.store_scatter(out_v, [row_splat, pos], res)
        return carry

    lax.fori_loop(0, _ROWS, row_body, 0)
    pltpu.sync_copy(out_v, out_hbm.at[pl.ds(base, _ROWS)])


@jax.jit
def _run(seg, spat, pack_flat, perm_flat):
    mesh = plsc.VectorSubcoreMesh(core_axis_name="c", subcore_axis_name="s")
    fn = functools.partial(
        pl.kernel,
        mesh=mesh,
        out_type=jax.ShapeDtypeStruct((_B, _N_OUT), jnp.float32),
        compiler_params=pltpu.CompilerParams(needs_layout_passes=False),
        scratch_types=[
            pltpu.VMEM((_ROWS, _N_SEG), jnp.float32),
            pltpu.VMEM((_ROWS, _N_SPAT), jnp.float32),
            pltpu.VMEM((_ROWS * _SEG_W,), jnp.float32),
            pltpu.VMEM((_ROWS * _SPAT_W,), jnp.float32),
            pltpu.VMEM((_N_SLOTS * 16,), jnp.int32),
            pltpu.VMEM((_N_GROUPS * 16,), jnp.int32),
            pltpu.VMEM((_ROWS, _N_OUT), jnp.float32),
        ],
    )(_sc_body)
    return fn(seg, spat, pack_flat, perm_flat)


def kernel(seg_confidences, spatial_features, rule_weights):
    del rule_weights
    return _run(seg_confidences, spatial_features,
                jnp.asarray(_PACK_TAB).reshape(-1),
                jnp.asarray(_PERM_TAB).reshape(-1))
